# baseline (speedup 1.0000x reference)
import functools
import math

import jax
import jax.numpy as jnp
from jax import lax
from jax.experimental import pallas as pl
from jax.experimental.pallas import tpu as pltpu
from jax.experimental.pallas import tpu_sc as plsc

N_NODES = 100000
N_EDGES = 1600000
HEADS = 4
D_HEAD = 8
INV_SQRT_D = 1.0 / math.sqrt(D_HEAD)

NC = 2
NS = 16
NW = NC * NS

CHUNK = 400
GROUPS = CHUNK // 16
E_PER_W = N_EDGES // NW
P1_CHUNKS = E_PER_W // CHUNK
E_PER_S = N_EDGES // NS
P2_CHUNKS = E_PER_S // CHUNK

ROWS_PER_S = 6256
N_PAD = ROWS_PER_S * NS
DW_PER_W = N_PAD * HEADS // NW

_mesh = plsc.VectorSubcoreMesh(core_axis_name="c", subcore_axis_name="s")
_params = pltpu.CompilerParams(
    needs_layout_passes=False, use_tc_tiling_on_sc=False
)


def _iota16():
    return lax.iota(jnp.int32, 16)


def _bc(x):
    return jnp.full((16,), x, jnp.int32)


@functools.partial(
    pl.kernel,
    out_type=(
        jax.ShapeDtypeStruct((N_EDGES, HEADS), jnp.float32),
        jax.ShapeDtypeStruct((NC * N_PAD, HEADS), jnp.float32),
    ),
    mesh=_mesh,
    compiler_params=_params,
    scratch_types=[
        pltpu.VMEM((CHUNK,), jnp.int32),
        pltpu.VMEM((CHUNK, 32), jnp.float32),
        pltpu.VMEM((CHUNK * 24,), jnp.float32),
        pltpu.VMEM((CHUNK * 8,), jnp.float32),
        pltpu.VMEM((CHUNK, HEADS), jnp.float32),
        pltpu.VMEM_SHARED((N_PAD, HEADS), jnp.float32),
        pltpu.SemaphoreType.DMA,
        pltpu.SemaphoreType.DMA,
    ],
)
def _phase1(kv, ks, qall, dsts, z4, s_out, dpart,
            idx_v, qrows, kflat, ksflat, sbuf, den_sh, sem, sem2):
    c = lax.axis_index("c")
    sid = lax.axis_index("s")
    wid = sid * NC + c
    r0 = sid * ROWS_PER_S

    pltpu.sync_copy(z4.at[pl.ds(r0, ROWS_PER_S)],
                    den_sh.at[pl.ds(r0, ROWS_PER_S)])
    plsc.subcore_barrier()

    iota24 = _iota16() * 24
    iota8 = _iota16() * 8

    def chunk_body(i, carry):
        e0 = wid * E_PER_W + i * CHUNK
        pltpu.sync_copy(dsts.at[pl.ds(e0, CHUNK)], idx_v)
        gq = pltpu.async_copy(qall.at[idx_v], qrows, sem)
        gk = pltpu.async_copy(kv.at[pl.ds(e0 * 24, CHUNK * 24)], kflat, sem2)
        gs = pltpu.async_copy(ks.at[pl.ds(e0 * 8, CHUNK * 8)], ksflat, sem2)
        gk.wait()
        gs.wait()
        gq.wait()

        def group_body(g, gcarry):
            rows = _iota16() + g * 16
            for h in range(HEADS):
                acc = jnp.zeros((16,), jnp.float32)
                for j in range(6):
                    kc = plsc.load_gather(
                        kflat, [iota24 + g * 384 + (6 * h + j)])
                    qc = plsc.load_gather(qrows, [rows, _bc(8 * h + j)])
                    acc = acc + kc * qc
                for j in range(2):
                    kc = plsc.load_gather(
                        ksflat, [iota8 + g * 128 + (2 * h + j)])
                    qc = plsc.load_gather(qrows, [rows, _bc(8 * h + 6 + j)])
                    acc = acc + kc * qc
                s = jnp.exp(acc * INV_SQRT_D)
                plsc.store_scatter(sbuf, [rows, _bc(h)], s)
            return gcarry

        lax.fori_loop(0, GROUPS, group_body, 0)
        pltpu.sync_copy(sbuf, s_out.at[pl.ds(e0, CHUNK)])
        pltpu.sync_copy(sbuf, den_sh.at[idx_v], add=True)
        return carry

    lax.fori_loop(0, P1_CHUNKS, chunk_body, 0)

    plsc.subcore_barrier()
    pltpu.sync_copy(den_sh.at[pl.ds(r0, ROWS_PER_S)],
                    dpart.at[pl.ds(c * N_PAD + r0, ROWS_PER_S)])


@functools.partial(
    pl.kernel,
    out_type=jax.ShapeDtypeStruct((N_PAD * HEADS,), jnp.float32),
    mesh=_mesh,
    compiler_params=_params,
    scratch_types=[
        pltpu.VMEM((DW_PER_W,), jnp.float32),
        pltpu.VMEM((DW_PER_W,), jnp.float32),
        pltpu.VMEM((DW_PER_W,), jnp.float32),
    ],
)
def _combine(dflat, dtot, a, b, o):
    c = lax.axis_index("c")
    sid = lax.axis_index("s")
    wid = sid * NC + c
    w0 = wid * DW_PER_W
    pltpu.sync_copy(dflat.at[pl.ds(w0, DW_PER_W)], a)
    pltpu.sync_copy(dflat.at[pl.ds(N_PAD * HEADS + w0, DW_PER_W)], b)

    def body(i, carry):
        sl = pl.ds(i * 16, 16)
        o[sl] = a[sl] + b[sl]
        return carry

    lax.fori_loop(0, DW_PER_W // 16, body, 0)
    pltpu.sync_copy(o, dtot.at[pl.ds(w0, DW_PER_W)])


@functools.partial(
    pl.kernel,
    out_type=jax.ShapeDtypeStruct((NC * N_PAD, 16), jnp.float32),
    mesh=_mesh,
    compiler_params=_params,
    scratch_types=[
        pltpu.VMEM((CHUNK,), jnp.int32),
        pltpu.VMEM((CHUNK * 4,), jnp.float32),
        pltpu.VMEM((CHUNK, HEADS), jnp.float32),
        pltpu.VMEM((CHUNK * 24,), jnp.float32),
        pltpu.VMEM((CHUNK * 8,), jnp.float32),
        pltpu.VMEM((CHUNK, 16), jnp.float32),
        pltpu.VMEM_SHARED((N_PAD, 16), jnp.float32),
        pltpu.SemaphoreType.DMA,
        pltpu.SemaphoreType.DMA,
    ],
)
def _phase2(s_hbm, dtoth, dsts, vv, vs, z16, accout,
            idx_v, sflat, db, vvflat, vsflat, msg, acc_sh, sem, sem2):
    c = lax.axis_index("c")
    sid = lax.axis_index("s")
    r0 = sid * ROWS_PER_S

    pltpu.sync_copy(z16.at[pl.ds(r0, ROWS_PER_S)],
                    acc_sh.at[pl.ds(r0, ROWS_PER_S)])
    plsc.subcore_barrier()

    iota24 = _iota16() * 24
    iota8 = _iota16() * 8
    iota4 = _iota16() * 4

    def chunk_body(i, carry):
        e0 = sid * E_PER_S + i * CHUNK
        pltpu.sync_copy(dsts.at[pl.ds(e0, CHUNK)], idx_v)
        g0 = pltpu.async_copy(dtoth.at[idx_v], db, sem)
        g1 = pltpu.async_copy(s_hbm.at[pl.ds(e0 * 4, CHUNK * 4)], sflat, sem2)
        g2 = pltpu.async_copy(vv.at[pl.ds(e0 * 24, CHUNK * 24)], vvflat, sem2)
        g3 = pltpu.async_copy(vs.at[pl.ds(e0 * 8, CHUNK * 8)], vsflat, sem2)
        g1.wait()
        g2.wait()
        g3.wait()
        g0.wait()

        def group_body(g, gcarry):
            rows = _iota16() + g * 16
            for l in range(2):
                h = 2 * c + l
                sv = plsc.load_gather(sflat, [iota4 + g * 64 + h])
                den = plsc.load_gather(db, [rows, _bc(0) + h])
                w = sv / den
                for j in range(6):
                    vc = plsc.load_gather(
                        vvflat, [iota24 + g * 384 + (6 * l + j) + 12 * c])
                    plsc.store_scatter(msg, [rows, _bc(6 * l + j)], vc * w)
                for j in range(2):
                    vc = plsc.load_gather(
                        vsflat, [iota8 + g * 128 + (2 * l + j) + 4 * c])
                    plsc.store_scatter(msg, [rows, _bc(12 + 2 * l + j)], vc * w)
            return gcarry

        lax.fori_loop(0, GROUPS, group_body, 0)
        pltpu.sync_copy(msg, acc_sh.at[idx_v], add=True)
        return carry

    lax.fori_loop(0, P2_CHUNKS, chunk_body, 0)

    plsc.subcore_barrier()
    pltpu.sync_copy(acc_sh.at[pl.ds(r0, ROWS_PER_S)],
                    accout.at[pl.ds(c * N_PAD + r0, ROWS_PER_S)])


def kernel(q_vec, q_scalar, k_vec, k_scalar, v_vec, v_scalar, edge_index):
    N, E, H = N_NODES, N_EDGES, HEADS

    qall = jnp.concatenate(
        [q_vec.reshape(N, H, 6), q_scalar.reshape(N, H, 2)], axis=-1
    ).reshape(N, H * 8)
    kv = k_vec.reshape(E * 24)
    ks = k_scalar.reshape(E * 8)
    vv = v_vec.reshape(E * 24)
    vs = v_scalar.reshape(E * 8)
    dsts = edge_index[1].astype(jnp.int32)

    z4 = jnp.zeros((N_PAD, H), jnp.float32)
    z16 = jnp.zeros((N_PAD, 16), jnp.float32)

    s, dpart = _phase1(kv, ks, qall, dsts, z4)
    dtot = _combine(dpart.reshape(NC * N_PAD * H)).reshape(N_PAD, H)

    acc = _phase2(s.reshape(E * H), dtot, dsts, vv, vs, z16)
    a0 = acc[:N]
    a1 = acc[N_PAD:N_PAD + N]

    out_vec = jnp.concatenate([a0[:, :12], a1[:, :12]], axis=1).reshape(N, 8, 3)
    out_scalar = jnp.concatenate([a0[:, 12:16], a1[:, 12:16]], axis=1).reshape(N, 8, 1)
    return (out_vec, out_scalar)


# speedup vs baseline: 4.1015x; 4.1015x over previous
import functools
import math

import jax
import jax.numpy as jnp
from jax import lax
from jax.experimental import pallas as pl
from jax.experimental.pallas import tpu as pltpu
from jax.experimental.pallas import tpu_sc as plsc

N_NODES = 100000
N_EDGES = 1600000
HEADS = 4
D_HEAD = 8
INV_SQRT_D = 1.0 / math.sqrt(D_HEAD)

NC = 2
NS = 16
NW = NC * NS

CHUNK = 80
E_PER_W = N_EDGES // NW
P1_CHUNKS = E_PER_W // CHUNK
E_PER_S = N_EDGES // NS
P2_CHUNKS = E_PER_S // CHUNK

ROWS_PER_S = 6256
N_PAD = ROWS_PER_S * NS

_mesh = plsc.VectorSubcoreMesh(core_axis_name="c", subcore_axis_name="s")


def _iota16():
    return lax.iota(jnp.int32, 16)


def _bc(x):
    return jnp.full((16,), x, jnp.int32)


@functools.partial(
    pl.kernel,
    out_type=(
        jax.ShapeDtypeStruct((N_EDGES, HEADS), jnp.float32),
        jax.ShapeDtypeStruct((NC * N_PAD, HEADS), jnp.float32),
    ),
    mesh=_mesh,
    compiler_params=pltpu.CompilerParams(needs_layout_passes=False, use_tc_tiling_on_sc=False),
    scratch_types=[
        pltpu.VMEM((CHUNK,), jnp.int32),
        pltpu.VMEM((CHUNK, 32), jnp.float32),
        pltpu.VMEM((CHUNK, 24), jnp.float32),
        pltpu.VMEM((CHUNK, 8), jnp.float32),
        pltpu.VMEM((CHUNK, HEADS), jnp.float32),
        pltpu.VMEM_SHARED((N_PAD, HEADS), jnp.float32),
        pltpu.SemaphoreType.DMA,
    ],
)
def _phase1(kv, ks, qall, dsts, z4, s_out, dpart,
            idx_v, qrows, key24, ks8, sbuf, den_sh, sem):
    c = lax.axis_index("c")
    sid = lax.axis_index("s")
    wid = sid * NC + c
    r0 = sid * ROWS_PER_S

    pltpu.sync_copy(z4.at[pl.ds(r0, ROWS_PER_S)],
                    den_sh.at[pl.ds(r0, ROWS_PER_S)])
    plsc.subcore_barrier()

    def chunk_body(i, carry):
        e0 = wid * E_PER_W + i * CHUNK
        pltpu.sync_copy(dsts.at[pl.ds(e0, CHUNK)], idx_v)
        gq = pltpu.async_copy(qall.at[idx_v], qrows, sem)
        pltpu.sync_copy(kv.at[pl.ds(e0, CHUNK)], key24)
        pltpu.sync_copy(ks.at[pl.ds(e0, CHUNK)], ks8)
        gq.wait()
        for g in range(CHUNK // 16):
            rows = _iota16() + g * 16
            for h in range(HEADS):
                acc = jnp.zeros((16,), jnp.float32)
                for j in range(6):
                    kc = plsc.load_gather(key24, [rows, _bc(6 * h + j)])
                    qc = plsc.load_gather(qrows, [rows, _bc(8 * h + j)])
                    acc = acc + kc * qc
                for j in range(2):
                    kc = plsc.load_gather(ks8, [rows, _bc(2 * h + j)])
                    qc = plsc.load_gather(qrows, [rows, _bc(8 * h + 6 + j)])
                    acc = acc + kc * qc
                s = jnp.exp(acc * INV_SQRT_D)
                plsc.store_scatter(sbuf, [rows, _bc(h)], s)
        pltpu.sync_copy(sbuf, s_out.at[pl.ds(e0, CHUNK)])
        pltpu.sync_copy(sbuf, den_sh.at[idx_v], add=True)
        return carry

    lax.fori_loop(0, P1_CHUNKS, chunk_body, 0)

    plsc.subcore_barrier()
    pltpu.sync_copy(den_sh.at[pl.ds(r0, ROWS_PER_S)],
                    dpart.at[pl.ds(c * N_PAD + r0, ROWS_PER_S)])


@functools.partial(
    pl.kernel,
    out_type=jax.ShapeDtypeStruct((NC * N_PAD, 16), jnp.float32),
    mesh=_mesh,
    compiler_params=pltpu.CompilerParams(needs_layout_passes=False, use_tc_tiling_on_sc=False),
    scratch_types=[
        pltpu.VMEM((CHUNK,), jnp.int32),
        pltpu.VMEM((CHUNK, HEADS), jnp.float32),
        pltpu.VMEM((CHUNK, HEADS), jnp.float32),
        pltpu.VMEM((CHUNK, HEADS), jnp.float32),
        pltpu.VMEM((CHUNK, 24), jnp.float32),
        pltpu.VMEM((CHUNK, 8), jnp.float32),
        pltpu.VMEM((CHUNK, 16), jnp.float32),
        pltpu.VMEM_SHARED((N_PAD, 16), jnp.float32),
        pltpu.SemaphoreType.DMA,
    ],
)
def _phase2(s_hbm, d0h, d1h, dsts, vv, vs, z16, accout,
            idx_v, schunk, d0b, d1b, vvbuf, vsbuf, msg, acc_sh, sem):
    c = lax.axis_index("c")
    sid = lax.axis_index("s")
    r0 = sid * ROWS_PER_S

    pltpu.sync_copy(z16.at[pl.ds(r0, ROWS_PER_S)],
                    acc_sh.at[pl.ds(r0, ROWS_PER_S)])
    plsc.subcore_barrier()

    def chunk_body(i, carry):
        e0 = sid * E_PER_S + i * CHUNK
        pltpu.sync_copy(dsts.at[pl.ds(e0, CHUNK)], idx_v)
        g0 = pltpu.async_copy(d0h.at[idx_v], d0b, sem)
        g1 = pltpu.async_copy(d1h.at[idx_v], d1b, sem)
        pltpu.sync_copy(s_hbm.at[pl.ds(e0, CHUNK)], schunk)
        pltpu.sync_copy(vv.at[pl.ds(e0, CHUNK)], vvbuf)
        pltpu.sync_copy(vs.at[pl.ds(e0, CHUNK)], vsbuf)
        g0.wait()
        g1.wait()
        for g in range(CHUNK // 16):
            rows = _iota16() + g * 16
            for l in range(2):
                h = 2 * c + l
                sv = plsc.load_gather(schunk, [rows, _bc(h)])
                den = (plsc.load_gather(d0b, [rows, _bc(h)])
                       + plsc.load_gather(d1b, [rows, _bc(h)]))
                w = sv / den
                for j in range(6):
                    vc = plsc.load_gather(vvbuf, [rows, _bc(6 * h + j)])
                    plsc.store_scatter(msg, [rows, _bc(6 * l + j)], vc * w)
                for j in range(2):
                    vc = plsc.load_gather(vsbuf, [rows, _bc(2 * h + j)])
                    plsc.store_scatter(msg, [rows, _bc(12 + 2 * l + j)], vc * w)
        pltpu.sync_copy(msg, acc_sh.at[idx_v], add=True)
        return carry

    lax.fori_loop(0, P2_CHUNKS, chunk_body, 0)

    plsc.subcore_barrier()
    pltpu.sync_copy(acc_sh.at[pl.ds(r0, ROWS_PER_S)],
                    accout.at[pl.ds(c * N_PAD + r0, ROWS_PER_S)])


def kernel(q_vec, q_scalar, k_vec, k_scalar, v_vec, v_scalar, edge_index):
    N, E, H = N_NODES, N_EDGES, HEADS

    qall = jnp.concatenate(
        [q_vec.reshape(N, H, 6), q_scalar.reshape(N, H, 2)], axis=-1
    ).reshape(N, H * 8)
    kv = k_vec.reshape(E, 24)
    ks = k_scalar.reshape(E, 8)
    vv = v_vec.reshape(E, 24)
    vs = v_scalar.reshape(E, 8)
    dsts = edge_index[1].astype(jnp.int32)

    z4 = jnp.zeros((N_PAD, H), jnp.float32)
    z16 = jnp.zeros((N_PAD, 16), jnp.float32)

    s, dpart = _phase1(kv, ks, qall, dsts, z4)
    d0h = dpart[:N_PAD]
    d1h = dpart[N_PAD:]

    acc = _phase2(s, d0h, d1h, dsts, vv, vs, z16)
    a0 = acc[:N]
    a1 = acc[N_PAD:N_PAD + N]

    out_vec = jnp.concatenate([a0[:, :12], a1[:, :12]], axis=1).reshape(N, 8, 3)
    out_scalar = jnp.concatenate([a0[:, 12:16], a1[:, 12:16]], axis=1).reshape(N, 8, 1)
    return (out_vec, out_scalar)
